# baseline (speedup 1.0000x reference)
import jax
import jax.numpy as jnp
from jax import lax
from jax.experimental import pallas as pl
from jax.experimental.pallas import tpu as pltpu
from jax.experimental.pallas import tpu_sc as plsc

H = 224
HH = 112
NB = 1536
NW = 32
BPW = NB // NW
NCHUNK = 2 * BPW
NBUF = 4


def _sc_body(x_hbm, idx_hbm, out_hbm, idx_v, t0, t1, shared, *sems):
    sid = lax.axis_index("s")
    wid = sid * 2 + lax.axis_index("c")
    base = wid * BPW
    pltpu.sync_copy(idx_hbm.at[pl.ds(base, BPW)], idx_v)

    gsems = sems[:NBUF]
    ssems = sems[NBUF:]

    def src_row(i):
        return idx_v[pl.ds((i // 16) * 16, 16)][i % 16]

    def buf(j):
        k = j % NBUF
        if k == 0:
            return t0
        if k == 1:
            return t1
        return shared.at[sid, k - 2]

    def hbm_slice(ref, j):
        i, half = j // 2, j % 2
        return ref.at[pl.ds(i, 1), pl.ds(half * HH, HH)]

    def start_gather(j):
        i = j // 2
        src = x_hbm.at[pl.ds(src_row(i), 1), pl.ds((j % 2) * HH, HH)]
        pltpu.async_copy(src, buf(j), gsems[j % NBUF])

    def wait_gather(j):
        pltpu.make_async_copy(
            x_hbm.at[pl.ds(0, 1), pl.ds(0, HH)], buf(j), gsems[j % NBUF]
        ).wait()

    def start_store(j):
        i, half = j // 2, j % 2
        dst = out_hbm.at[pl.ds(base + i, 1), pl.ds(half * HH, HH)]
        pltpu.async_copy(buf(j), dst, ssems[j % NBUF])

    def wait_store(j):
        pltpu.make_async_copy(
            buf(j), out_hbm.at[pl.ds(base, 1), pl.ds(0, HH)], ssems[j % NBUF]
        ).wait()

    start_gather(0)
    start_gather(1)
    start_gather(2)
    for j in range(NCHUNK):
        wait_gather(j)
        start_store(j)
        if j + 3 < NCHUNK:
            if j >= 1:
                wait_store(j - 1)
            start_gather(j + 3)
    for j in range(NCHUNK - 4, NCHUNK):
        wait_store(j)


def kernel(x, permutation):
    b, c, h, w = x.shape
    xr = x.reshape(NB, H, H)
    idx = (
        jnp.arange(b, dtype=jnp.int32)[:, None] * c
        + permutation.astype(jnp.int32)[None, :]
    ).reshape(NB)
    mesh = plsc.VectorSubcoreMesh(core_axis_name="c", subcore_axis_name="s")
    out = pl.kernel(
        _sc_body,
        mesh=mesh,
        out_type=jax.ShapeDtypeStruct((NB, H, H), x.dtype),
        scratch_types=[
            pltpu.VMEM((BPW,), jnp.int32),
            pltpu.VMEM((1, HH, H), jnp.float32),
            pltpu.VMEM((1, HH, H), jnp.float32),
            pltpu.VMEM_SHARED((16, 2, 1, HH, H), jnp.float32),
        ]
        + [pltpu.SemaphoreType.DMA] * (2 * NBUF),
    )(xr, idx)
    return out.reshape(b, c, h, w)


# speedup vs baseline: 1.0396x; 1.0396x over previous
import jax
import jax.numpy as jnp
from jax import lax
from jax.experimental import pallas as pl
from jax.experimental.pallas import tpu as pltpu
from jax.experimental.pallas import tpu_sc as plsc

H = 224
HH = 112
NB = 1536
NW = 32
BPW = NB // NW
NCHUNK = 2 * BPW
NBUF = 4


def _sc_body(x_hbm, idx_hbm, out_hbm, idx_v, shared, *sems):
    sid = lax.axis_index("s")
    wid = sid * 2 + lax.axis_index("c")
    base = wid * BPW
    pltpu.sync_copy(idx_hbm.at[pl.ds(base, BPW)], idx_v)

    gsems = sems[:NBUF]
    ssems = sems[NBUF:]

    def src_row(i):
        return idx_v[pl.ds((i // 16) * 16, 16)][i % 16]

    def buf(j):
        return shared.at[sid, j % NBUF]

    def hbm_slice(ref, j):
        i, half = j // 2, j % 2
        return ref.at[pl.ds(i, 1), pl.ds(half * HH, HH)]

    def start_gather(j):
        i = j // 2
        src = x_hbm.at[pl.ds(src_row(i), 1), pl.ds((j % 2) * HH, HH)]
        pltpu.async_copy(src, buf(j), gsems[j % NBUF])

    def wait_gather(j):
        pltpu.make_async_copy(
            x_hbm.at[pl.ds(0, 1), pl.ds(0, HH)], buf(j), gsems[j % NBUF]
        ).wait()

    def start_store(j):
        i, half = j // 2, j % 2
        dst = out_hbm.at[pl.ds(base + i, 1), pl.ds(half * HH, HH)]
        pltpu.async_copy(buf(j), dst, ssems[j % NBUF])

    def wait_store(j):
        pltpu.make_async_copy(
            buf(j), out_hbm.at[pl.ds(base, 1), pl.ds(0, HH)], ssems[j % NBUF]
        ).wait()

    start_gather(0)
    start_gather(1)
    start_gather(2)
    for j in range(NCHUNK):
        wait_gather(j)
        start_store(j)
        if j + 3 < NCHUNK:
            if j >= 1:
                wait_store(j - 1)
            start_gather(j + 3)
    for j in range(NCHUNK - 4, NCHUNK):
        wait_store(j)


def kernel(x, permutation):
    b, c, h, w = x.shape
    xr = x.reshape(NB, H, H)
    idx = (
        jnp.arange(b, dtype=jnp.int32)[:, None] * c
        + permutation.astype(jnp.int32)[None, :]
    ).reshape(NB)
    mesh = plsc.VectorSubcoreMesh(core_axis_name="c", subcore_axis_name="s")
    out = pl.kernel(
        _sc_body,
        mesh=mesh,
        out_type=jax.ShapeDtypeStruct((NB, H, H), x.dtype),
        scratch_types=[
            pltpu.VMEM((BPW,), jnp.int32),
            pltpu.VMEM_SHARED((16, NBUF, 1, HH, H), jnp.float32),
        ]
        + [pltpu.SemaphoreType.DMA] * (2 * NBUF),
    )(xr, idx)
    return out.reshape(b, c, h, w)
